# baseline (speedup 1.0000x reference)
import functools

import jax
import jax.numpy as jnp
from jax import lax
from jax.experimental import pallas as pl
from jax.experimental.pallas import tpu as pltpu
from jax.experimental.pallas import tpu_sc as plsc

_W = 16
_L = 16
_NC = 2
_NS = 16
_NW = _NC * _NS
_R = 32
_NBUF = 3
_KSPLIT = 2


def _make_sc_add(rows, d):
    rpw = rows // _NW
    nchunk = rpw // _R
    mesh = plsc.VectorSubcoreMesh(core_axis_name="c", subcore_axis_name="s")

    @functools.partial(
        pl.kernel,
        mesh=mesh,
        out_type=jax.ShapeDtypeStruct((rows, d), jnp.float32),
        scratch_types=[
            pltpu.VMEM((_W, d), jnp.float32),
        ] + [pltpu.VMEM((_R, d), jnp.float32)] * _NBUF
          + [pltpu.SemaphoreType.DMA] * (2 * _NBUF),
    )
    def sc_add(x_hbm, t_hbm, o_hbm, tab, b0, b1, b2, si0, si1, si2, so0, so1, so2):
        wid = lax.axis_index("s") * _NC + lax.axis_index("c")
        base = wid * rpw
        bufs = (b0, b1, b2)
        sis = (si0, si1, si2)
        sos = (so0, so1, so2)

        pltpu.sync_copy(t_hbm, tab)

        hr = _R // _KSPLIT

        class _Cp:
            def __init__(self, g, b, out):
                self.g, self.b, self.out = g, b, out

            def _piece(self, h, n):
                row = base + self.g * _R + h * hr
                if self.out:
                    return pltpu.make_async_copy(
                        bufs[self.b].at[pl.ds(h * hr, n)],
                        o_hbm.at[pl.ds(row, n)], sos[self.b])
                return pltpu.make_async_copy(
                    x_hbm.at[pl.ds(row, n)],
                    bufs[self.b].at[pl.ds(h * hr, n)], sis[self.b])

            def start(self):
                for h in range(_KSPLIT):
                    self._piece(h, hr).start()

            def wait(self):
                self._piece(0, _R).wait()

        def cin(g, b):
            return _Cp(g, b, out=False)

        def cout(g, b):
            return _Cp(g, b, out=True)

        def compute(b):
            buf = bufs[b]

            @plsc.parallel_loop(0, d // _L, 1)
            def jbody(j):
                s = pl.ds(j * _L, _L)
                ts = [tab[k, s] for k in range(_W)]
                for k in range(_R):
                    buf[k, s] = buf[k, s] + ts[k % _W]

        def step(g, b, wait_out, start_in):
            cin(g, b).wait()
            if wait_out:
                cout(g - 2, (b + 1) % _NBUF).wait()
            if start_in:
                cin(g + 1, (b + 1) % _NBUF).start()
            compute(b)
            cout(g, b).start()

        cin(0, 0).start()
        step(0, 0, wait_out=False, start_in=True)
        step(1, 1, wait_out=False, start_in=True)
        step(2, 2, wait_out=True, start_in=True)

        def gbody(i, c):
            g0 = 3 * i
            for b in range(_NBUF):
                g = g0 + b
                step(g, b, wait_out=True, start_in=True)
            return c

        lax.fori_loop(1, nchunk // 3, gbody, 0)

        step(nchunk - 2, (nchunk - 2) % _NBUF, wait_out=True, start_in=True)
        step(nchunk - 1, (nchunk - 1) % _NBUF, wait_out=True, start_in=False)
        for g in (nchunk - 2, nchunk - 1):
            cout(g, g % _NBUF).wait()

    return sc_add


def kernel(visn_feats, table):
    B, S, D = visn_feats.shape
    rows = B * S
    x2 = visn_feats.reshape(rows, D)
    out = _make_sc_add(rows, D)(x2, table)
    return out.reshape(B, S, D)
